# baseline (speedup 1.0000x reference)
import functools

import jax
import jax.numpy as jnp
from jax import lax
from jax.experimental import pallas as pl
from jax.experimental.pallas import tpu as pltpu
from jax.experimental.pallas import tpu_sc as plsc

N = 10000
E = 320000
D_IN = 128
H2 = 64
NCLS = 16

_NC = 2
_NS = 16
_NW = _NC * _NS
_EPW = E // _NW
_CH = 112
_STEPS = 90
_EPAD = _STEPS * _CH - _EPW
_PAIRS = _STEPS // 2
_DGR = 80
_TRASH = N + _DGR
_NROW = N + _DGR + 8
_ROWS_A = 624


def _make_seg_sum(d_feat, with_deg):
    mesh = plsc.VectorSubcoreMesh(core_axis_name="c", subcore_axis_name="s")
    nrow = _NROW
    tail = nrow - _NS * _ROWS_A
    scratch = [
        pltpu.VMEM_SHARED((nrow, d_feat), jnp.float32),
        pltpu.VMEM((_STEPS, _CH), jnp.int32),
        pltpu.VMEM((2, _CH), jnp.int32),
        pltpu.VMEM((_CH, d_feat), jnp.float32),
        pltpu.VMEM((_CH, d_feat), jnp.float32),
        pltpu.SemaphoreType.DMA,
        pltpu.SemaphoreType.DMA,
        pltpu.SemaphoreType.DMA,
        pltpu.SemaphoreType.DMA,
    ]
    if with_deg:
        scratch += [
            pltpu.VMEM((_DGR, d_feat), jnp.float32),
            pltpu.VMEM((_DGR,), jnp.int32),
        ]

    @functools.partial(
        pl.kernel,
        mesh=mesh,
        compiler_params=pltpu.CompilerParams(
            use_tc_tiling_on_sc=False, needs_layout_passes=False),
        out_type=jax.ShapeDtypeStruct((_NC, nrow, d_feat), jnp.float32),
        scratch_types=scratch,
    )
    def seg_sum(feat_hbm, src_hbm, dst_hbm, zeros_hbm, degidx_hbm, out_hbm,
                acc_sh, dsts_v, sidx_v, buf_a, buf_b, sem_a, sem_b,
                sem_ia, sem_ib, *deg_scratch):
        cid = lax.axis_index("c")
        sid = lax.axis_index("s")
        wid = sid * _NC + cid

        base_r = pl.multiple_of(sid * _ROWS_A, 8)
        pltpu.sync_copy(zeros_hbm.at[pl.ds(base_r, _ROWS_A)],
                        acc_sh.at[pl.ds(base_r, _ROWS_A)])

        @pl.when(sid == _NS - 1)
        def _():
            t0 = pl.multiple_of(_NS * _ROWS_A, 8)
            pltpu.sync_copy(zeros_hbm.at[pl.ds(t0, tail)],
                            acc_sh.at[pl.ds(t0, tail)])

        pltpu.sync_copy(dst_hbm.at[wid], dsts_v)
        pltpu.sync_copy(src_hbm.at[wid, 0], sidx_v.at[0])
        pltpu.sync_copy(src_hbm.at[wid, 1], sidx_v.at[1])
        if with_deg:
            deg_v, degidx_v = deg_scratch
            pltpu.sync_copy(zeros_hbm.at[pl.ds(0, _DGR)], deg_v)
            pltpu.sync_copy(degidx_hbm, degidx_v)
            ones16 = jnp.ones((16,), jnp.float32)

        plsc.subcore_barrier()

        pltpu.async_copy(feat_hbm.at[sidx_v.at[0]], buf_a, sem_a)
        pltpu.async_copy(feat_hbm.at[sidx_v.at[1]], buf_b, sem_b)

        def count_deg(i):
            for j in range(_CH // 16):
                dv = dsts_v[i, pl.ds(16 * j, 16)]
                row = lax.shift_right_logical(dv, 7)
                col = lax.bitwise_and(dv, 127)
                plsc.addupdate_scatter(deg_v, [row, col], ones16)

        def half(i, buf, sem_g, slot, sem_i):
            pltpu.make_async_copy(feat_hbm.at[sidx_v.at[slot]], buf, sem_g).wait()

            @pl.when(i + 2 < _STEPS)
            def _():
                pltpu.async_copy(src_hbm.at[wid, i + 2], sidx_v.at[slot], sem_i)

            pltpu.sync_copy(buf, acc_sh.at[dsts_v.at[i]], add=True)
            if with_deg:
                count_deg(i)

            @pl.when(i + 2 < _STEPS)
            def _():
                pltpu.make_async_copy(src_hbm.at[wid, i + 2], sidx_v.at[slot],
                                      sem_i).wait()
                pltpu.async_copy(feat_hbm.at[sidx_v.at[slot]], buf, sem_g)

        def pair(k, carry):
            i0 = 2 * k
            half(i0, buf_a, sem_a, 0, sem_ia)
            half(i0 + 1, buf_b, sem_b, 1, sem_ib)
            return carry

        lax.fori_loop(0, _PAIRS, pair, 0)
        if with_deg:
            pltpu.sync_copy(deg_v, acc_sh.at[degidx_v], add=True)

        plsc.subcore_barrier()

        pltpu.sync_copy(acc_sh.at[pl.ds(base_r, _ROWS_A)],
                        out_hbm.at[cid, pl.ds(base_r, _ROWS_A)])

        @pl.when(sid == _NS - 1)
        def _():
            t0 = pl.multiple_of(_NS * _ROWS_A, 8)
            pltpu.sync_copy(acc_sh.at[pl.ds(t0, tail)],
                            out_hbm.at[cid, pl.ds(t0, tail)])

    return seg_sum


_seg_sum_1 = _make_seg_sum(D_IN, True)
_seg_sum_2 = _make_seg_sum(H2, False)


_RB = 2000


def _layer1_body(msgp_ref, h_ref, deg_ref, w1_ref, b1_ref, w2_ref, y2_ref):
    msg = msgp_ref[0] + msgp_ref[1]
    agg = (msg + h_ref[...]) / (deg_ref[...] + 1.0)
    acc = jnp.dot(agg, w1_ref[...], preferred_element_type=jnp.float32)
    h1 = jnp.maximum(acc + b1_ref[...], 0.0)
    y2_ref[...] = jnp.dot(h1, w2_ref[...], preferred_element_type=jnp.float32)


def _layer1(msgp, x, deg, w1, b1, w2):
    return pl.pallas_call(
        _layer1_body,
        grid=(N // _RB,),
        in_specs=[
            pl.BlockSpec((_NC, _RB, D_IN), lambda i: (0, i, 0)),
            pl.BlockSpec((_RB, D_IN), lambda i: (i, 0)),
            pl.BlockSpec((_RB, 1), lambda i: (i, 0)),
            pl.BlockSpec((D_IN, D_IN), lambda i: (0, 0)),
            pl.BlockSpec((1, D_IN), lambda i: (0, 0)),
            pl.BlockSpec((D_IN, H2), lambda i: (0, 0)),
        ],
        out_specs=pl.BlockSpec((_RB, H2), lambda i: (i, 0)),
        out_shape=jax.ShapeDtypeStruct((N, H2), jnp.float32),
    )(msgp, x, deg, w1, b1.reshape(1, D_IN), w2)


def _cls_body(msgp_ref, y2_ref, deg_ref, b2_ref, wc_ref, bc_ref, o_ref):
    msg = msgp_ref[0] + msgp_ref[1]
    agg = (msg + y2_ref[...]) / (deg_ref[...] + 1.0)
    z = jnp.maximum(agg + b2_ref[...], 0.0)
    logits = jnp.dot(z, wc_ref[...], preferred_element_type=jnp.float32)
    logits = logits + bc_ref[...]
    m = jnp.max(logits, axis=-1, keepdims=True)
    e = jnp.exp(logits - m)
    o_ref[...] = e / jnp.sum(e, axis=-1, keepdims=True)


def _cls_layer(msgp, y2, deg, b2, wc, bc):
    return pl.pallas_call(
        _cls_body,
        grid=(N // _RB,),
        in_specs=[
            pl.BlockSpec((_NC, _RB, H2), lambda i: (0, i, 0)),
            pl.BlockSpec((_RB, H2), lambda i: (i, 0)),
            pl.BlockSpec((_RB, 1), lambda i: (i, 0)),
            pl.BlockSpec((1, H2), lambda i: (0, 0)),
            pl.BlockSpec((H2, NCLS), lambda i: (0, 0)),
            pl.BlockSpec((1, NCLS), lambda i: (0, 0)),
        ],
        out_specs=pl.BlockSpec((_RB, NCLS), lambda i: (i, 0)),
        out_shape=jax.ShapeDtypeStruct((N, NCLS), jnp.float32),
    )(msgp, y2, deg, b2.reshape(1, H2), wc, bc.reshape(1, NCLS))


_BM = 400


def _dec_body(a_ref, b_ref, o_ref):
    o_ref[...] = lax.dot_general(
        a_ref[...], b_ref[...],
        (((1,), (1,)), ((), ())),
        preferred_element_type=jnp.float32)


def _decoder(preds):
    return pl.pallas_call(
        _dec_body,
        grid=(N // _BM,),
        in_specs=[
            pl.BlockSpec((_BM, NCLS), lambda i: (i, 0)),
            pl.BlockSpec((N, NCLS), lambda i: (0, 0)),
        ],
        out_specs=pl.BlockSpec((_BM, N), lambda i: (i, 0)),
        out_shape=jax.ShapeDtypeStruct((N, N), jnp.float32),
    )(preds, preds)


def kernel(x, edge_index, W1, b1, W2, b2, Wc, bc):
    src = jnp.concatenate(
        [edge_index[0].astype(jnp.int32).reshape(_NW, _EPW),
         jnp.zeros((_NW, _EPAD), jnp.int32)], axis=1).reshape(_NW, _STEPS, _CH)
    dst = jnp.concatenate(
        [edge_index[1].astype(jnp.int32).reshape(_NW, _EPW),
         jnp.full((_NW, _EPAD), _TRASH, jnp.int32)], axis=1).reshape(_NW, _STEPS, _CH)
    zeros1 = jnp.zeros((_NROW, D_IN), jnp.float32)
    zeros2 = jnp.zeros((_NROW, H2), jnp.float32)
    degidx = jnp.arange(N, N + _DGR, dtype=jnp.int32)

    msgp1 = _seg_sum_1(x, src, dst, zeros1, degidx)
    deg = (msgp1[0, N:N + _DGR] + msgp1[1, N:N + _DGR]).reshape(-1)[:N].reshape(N, 1)
    y2 = _layer1(msgp1, x, deg, W1, b1, W2)

    msgp2 = _seg_sum_2(y2, src, dst, zeros2, degidx)
    preds = _cls_layer(msgp2, y2, deg, b2, Wc, bc)

    adj_hat = _decoder(preds)
    return preds, adj_hat


# speedup vs baseline: 1.1197x; 1.1197x over previous
import functools

import jax
import jax.numpy as jnp
from jax import lax
from jax.experimental import pallas as pl
from jax.experimental.pallas import tpu as pltpu
from jax.experimental.pallas import tpu_sc as plsc

N = 10000
E = 320000
D_IN = 128
H2 = 64
NCLS = 16

_NC = 2
_NS = 16
_NW = _NC * _NS
_EPW = E // _NW
_CH = 80
_STEPS = _EPW // _CH
_PAIRS = (_STEPS - 1) // 2
_DGR = 80
_NROW = N + _DGR
_ROWS_A = 624
_TAIL = _NROW - _NS * _ROWS_A


def _make_seg_sum(d_feat, with_deg):
    mesh = plsc.VectorSubcoreMesh(core_axis_name="c", subcore_axis_name="s")
    nrow = _NROW if with_deg else N
    tail = nrow - _NS * _ROWS_A
    scratch = [
        pltpu.VMEM_SHARED((nrow, d_feat), jnp.float32),
        pltpu.VMEM((_STEPS, _CH), jnp.int32),
        pltpu.VMEM((2, _CH), jnp.int32),
        pltpu.VMEM((_CH, d_feat), jnp.float32),
        pltpu.VMEM((_CH, d_feat), jnp.float32),
        pltpu.SemaphoreType.DMA,
        pltpu.SemaphoreType.DMA,
        pltpu.SemaphoreType.DMA,
        pltpu.SemaphoreType.DMA,
        pltpu.SemaphoreType.DMA,
        pltpu.SemaphoreType.DMA,
    ]
    if with_deg:
        scratch += [
            pltpu.VMEM((_DGR, d_feat), jnp.float32),
            pltpu.VMEM((_DGR,), jnp.int32),
        ]

    @functools.partial(
        pl.kernel,
        mesh=mesh,
        compiler_params=pltpu.CompilerParams(
            use_tc_tiling_on_sc=False, needs_layout_passes=False),
        out_type=jax.ShapeDtypeStruct((_NC, nrow, d_feat), jnp.float32),
        scratch_types=scratch,
    )
    def seg_sum(feat_hbm, src_hbm, dst_hbm, zeros_hbm, degidx_hbm, out_hbm,
                acc_sh, dsts_v, sidx_v, buf_a, buf_b, sem_a, sem_b,
                sem_ia, sem_ib, sem_sa, sem_sb, *deg_scratch):
        cid = lax.axis_index("c")
        sid = lax.axis_index("s")
        wid = sid * _NC + cid

        base_r = pl.multiple_of(sid * _ROWS_A, 8)
        pltpu.sync_copy(zeros_hbm.at[pl.ds(base_r, _ROWS_A)],
                        acc_sh.at[pl.ds(base_r, _ROWS_A)])

        @pl.when(sid == _NS - 1)
        def _():
            t0 = pl.multiple_of(_NS * _ROWS_A, 8)
            pltpu.sync_copy(zeros_hbm.at[pl.ds(t0, tail)],
                            acc_sh.at[pl.ds(t0, tail)])

        pltpu.sync_copy(dst_hbm.at[wid], dsts_v)
        pltpu.sync_copy(src_hbm.at[wid, 0], sidx_v.at[0])
        pltpu.sync_copy(src_hbm.at[wid, 1], sidx_v.at[1])
        if with_deg:
            deg_v, degidx_v = deg_scratch
            pltpu.sync_copy(zeros_hbm.at[pl.ds(0, _DGR)], deg_v)
            pltpu.sync_copy(degidx_hbm, degidx_v)
            ones16 = jnp.ones((16,), jnp.float32)

        plsc.subcore_barrier()

        def count_deg(i):
            for j in range(_CH // 16):
                dv = dsts_v[i, pl.ds(16 * j, 16)]
                row = lax.shift_right_logical(dv, 7)
                col = lax.bitwise_and(dv, 127)
                plsc.addupdate_scatter(deg_v, [row, col], ones16)

        def step(i, cur, oth):
            buf_c, sg_c, si_c, ss_c, slot_c = cur
            buf_o, sg_o, si_o, ss_o, slot_o = oth
            pltpu.make_async_copy(feat_hbm.at[sidx_v.at[slot_c]], buf_c, sg_c).wait()
            pltpu.async_copy(buf_c, acc_sh.at[dsts_v.at[i]], ss_c, add=True)

            @pl.when(i + 2 < _STEPS)
            def _():
                pltpu.async_copy(src_hbm.at[wid, i + 2], sidx_v.at[slot_c], si_c)

            if with_deg:
                count_deg(i)

            @pl.when(i + 1 < _STEPS)
            def _():
                pltpu.make_async_copy(buf_o, acc_sh.at[dsts_v.at[i - 1]],
                                      ss_o).wait()
                pltpu.make_async_copy(src_hbm.at[wid, i + 1], sidx_v.at[slot_o],
                                      si_o).wait()
                pltpu.async_copy(feat_hbm.at[sidx_v.at[slot_o]], buf_o, sg_o)

        ring_a = (buf_a, sem_a, sem_ia, sem_sa, 0)
        ring_b = (buf_b, sem_b, sem_ib, sem_sb, 1)

        pltpu.async_copy(feat_hbm.at[sidx_v.at[0]], buf_a, sem_a)
        pltpu.make_async_copy(feat_hbm.at[sidx_v.at[0]], buf_a, sem_a).wait()
        pltpu.async_copy(buf_a, acc_sh.at[dsts_v.at[0]], sem_sa, add=True)
        pltpu.async_copy(src_hbm.at[wid, 2], sidx_v.at[0], sem_ia)
        if with_deg:
            count_deg(0)
        pltpu.async_copy(feat_hbm.at[sidx_v.at[1]], buf_b, sem_b)

        def pair(k, carry):
            step(2 * k + 1, ring_b, ring_a)
            step(2 * k + 2, ring_a, ring_b)
            return carry

        lax.fori_loop(0, _PAIRS, pair, 0)
        pltpu.make_async_copy(buf_b, acc_sh.at[dsts_v.at[_STEPS - 2]],
                              sem_sb).wait()
        pltpu.make_async_copy(buf_a, acc_sh.at[dsts_v.at[_STEPS - 1]],
                              sem_sa).wait()
        if with_deg:
            pltpu.sync_copy(deg_v, acc_sh.at[degidx_v], add=True)

        plsc.subcore_barrier()

        pltpu.sync_copy(acc_sh.at[pl.ds(base_r, _ROWS_A)],
                        out_hbm.at[cid, pl.ds(base_r, _ROWS_A)])

        @pl.when(sid == _NS - 1)
        def _():
            t0 = pl.multiple_of(_NS * _ROWS_A, 8)
            pltpu.sync_copy(acc_sh.at[pl.ds(t0, tail)],
                            out_hbm.at[cid, pl.ds(t0, tail)])

    return seg_sum


_seg_sum_1 = _make_seg_sum(D_IN, True)
_seg_sum_2 = _make_seg_sum(H2, False)


_RB = 2000


def _layer1_body(msgp_ref, h_ref, deg_ref, w1_ref, b1_ref, w2_ref, y2_ref):
    msg = msgp_ref[0] + msgp_ref[1]
    agg = (msg + h_ref[...]) / (deg_ref[...] + 1.0)
    acc = jnp.dot(agg, w1_ref[...], preferred_element_type=jnp.float32)
    h1 = jnp.maximum(acc + b1_ref[...], 0.0)
    y2_ref[...] = jnp.dot(h1, w2_ref[...], preferred_element_type=jnp.float32)


def _layer1(msgp, x, deg, w1, b1, w2):
    return pl.pallas_call(
        _layer1_body,
        grid=(N // _RB,),
        in_specs=[
            pl.BlockSpec((_NC, _RB, D_IN), lambda i: (0, i, 0)),
            pl.BlockSpec((_RB, D_IN), lambda i: (i, 0)),
            pl.BlockSpec((_RB, 1), lambda i: (i, 0)),
            pl.BlockSpec((D_IN, D_IN), lambda i: (0, 0)),
            pl.BlockSpec((1, D_IN), lambda i: (0, 0)),
            pl.BlockSpec((D_IN, H2), lambda i: (0, 0)),
        ],
        out_specs=pl.BlockSpec((_RB, H2), lambda i: (i, 0)),
        out_shape=jax.ShapeDtypeStruct((N, H2), jnp.float32),
    )(msgp, x, deg, w1, b1.reshape(1, D_IN), w2)


def _cls_body(msgp_ref, y2_ref, deg_ref, b2_ref, wc_ref, bc_ref, o_ref):
    msg = msgp_ref[0] + msgp_ref[1]
    agg = (msg + y2_ref[...]) / (deg_ref[...] + 1.0)
    z = jnp.maximum(agg + b2_ref[...], 0.0)
    logits = jnp.dot(z, wc_ref[...], preferred_element_type=jnp.float32)
    logits = logits + bc_ref[...]
    m = jnp.max(logits, axis=-1, keepdims=True)
    e = jnp.exp(logits - m)
    o_ref[...] = e / jnp.sum(e, axis=-1, keepdims=True)


def _cls_layer(msgp, y2, deg, b2, wc, bc):
    return pl.pallas_call(
        _cls_body,
        grid=(N // _RB,),
        in_specs=[
            pl.BlockSpec((_NC, _RB, H2), lambda i: (0, i, 0)),
            pl.BlockSpec((_RB, H2), lambda i: (i, 0)),
            pl.BlockSpec((_RB, 1), lambda i: (i, 0)),
            pl.BlockSpec((1, H2), lambda i: (0, 0)),
            pl.BlockSpec((H2, NCLS), lambda i: (0, 0)),
            pl.BlockSpec((1, NCLS), lambda i: (0, 0)),
        ],
        out_specs=pl.BlockSpec((_RB, NCLS), lambda i: (i, 0)),
        out_shape=jax.ShapeDtypeStruct((N, NCLS), jnp.float32),
    )(msgp, y2, deg, b2.reshape(1, H2), wc, bc.reshape(1, NCLS))


_BM = 400


def _dec_body(a_ref, b_ref, o_ref):
    o_ref[...] = lax.dot_general(
        a_ref[...], b_ref[...],
        (((1,), (1,)), ((), ())),
        preferred_element_type=jnp.float32)


def _decoder(preds):
    return pl.pallas_call(
        _dec_body,
        grid=(N // _BM,),
        in_specs=[
            pl.BlockSpec((_BM, NCLS), lambda i: (i, 0)),
            pl.BlockSpec((N, NCLS), lambda i: (0, 0)),
        ],
        out_specs=pl.BlockSpec((_BM, N), lambda i: (i, 0)),
        out_shape=jax.ShapeDtypeStruct((N, N), jnp.float32),
    )(preds, preds)


def kernel(x, edge_index, W1, b1, W2, b2, Wc, bc):
    src = edge_index[0].astype(jnp.int32).reshape(_NW, _STEPS, _CH)
    dst = edge_index[1].astype(jnp.int32).reshape(_NW, _STEPS, _CH)
    zeros1 = jnp.zeros((_NROW, D_IN), jnp.float32)
    zeros2 = jnp.zeros((N, H2), jnp.float32)
    degidx = jnp.arange(N, N + _DGR, dtype=jnp.int32)

    msgp1 = _seg_sum_1(x, src, dst, zeros1, degidx)
    deg = (msgp1[0, N:] + msgp1[1, N:]).reshape(-1)[:N].reshape(N, 1)
    y2 = _layer1(msgp1, x, deg, W1, b1, W2)

    msgp2 = _seg_sum_2(y2, src, dst, zeros2, degidx)
    preds = _cls_layer(msgp2, y2, deg, b2, Wc, bc)

    adj_hat = _decoder(preds)
    return preds, adj_hat


# speedup vs baseline: 1.2811x; 1.1442x over previous
import functools

import jax
import jax.numpy as jnp
from jax import lax
from jax.experimental import pallas as pl
from jax.experimental.pallas import tpu as pltpu
from jax.experimental.pallas import tpu_sc as plsc

N = 10000
E = 320000
D_IN = 128
H2 = 64
NCLS = 16

_NC = 2
_NS = 16
_NW = _NC * _NS
_EPW = E // _NW
_CH = 80
_STEPS = _EPW // _CH
_PAIRS = (_STEPS - 1) // 2
_DGR = 80
_NROW = N + _DGR
_ROWS_A = 624
_TAIL = _NROW - _NS * _ROWS_A


def _make_seg_sum(d_feat, with_deg):
    mesh = plsc.VectorSubcoreMesh(core_axis_name="c", subcore_axis_name="s")
    nrow = _NROW if with_deg else N
    tail = nrow - _NS * _ROWS_A
    scratch = [
        pltpu.VMEM_SHARED((nrow, d_feat), jnp.float32),
        pltpu.VMEM((_STEPS, _CH), jnp.int32),
        pltpu.VMEM((2, _CH), jnp.int32),
        pltpu.VMEM((_CH, d_feat), jnp.float32),
        pltpu.VMEM((_CH, d_feat), jnp.float32),
        pltpu.SemaphoreType.DMA,
        pltpu.SemaphoreType.DMA,
        pltpu.SemaphoreType.DMA,
        pltpu.SemaphoreType.DMA,
    ]
    if with_deg:
        scratch += [
            pltpu.VMEM((_DGR, d_feat), jnp.float32),
            pltpu.VMEM((_DGR,), jnp.int32),
        ]

    @functools.partial(
        pl.kernel,
        mesh=mesh,
        compiler_params=pltpu.CompilerParams(
            use_tc_tiling_on_sc=False, needs_layout_passes=False),
        out_type=jax.ShapeDtypeStruct((_NC, nrow, d_feat), jnp.float32),
        scratch_types=scratch,
    )
    def seg_sum(feat_hbm, src_hbm, dst_hbm, zeros_hbm, degidx_hbm, out_hbm,
                acc_sh, dsts_v, sidx_v, buf_a, buf_b, sem_a, sem_b,
                sem_ia, sem_ib, *deg_scratch):
        cid = lax.axis_index("c")
        sid = lax.axis_index("s")
        wid = sid * _NC + cid

        base_r = pl.multiple_of(sid * _ROWS_A, 8)
        pltpu.sync_copy(zeros_hbm.at[pl.ds(base_r, _ROWS_A)],
                        acc_sh.at[pl.ds(base_r, _ROWS_A)])

        @pl.when(sid == _NS - 1)
        def _():
            t0 = pl.multiple_of(_NS * _ROWS_A, 8)
            pltpu.sync_copy(zeros_hbm.at[pl.ds(t0, tail)],
                            acc_sh.at[pl.ds(t0, tail)])

        pltpu.sync_copy(dst_hbm.at[wid], dsts_v)
        pltpu.sync_copy(src_hbm.at[wid, 0], sidx_v.at[0])
        pltpu.sync_copy(src_hbm.at[wid, 1], sidx_v.at[1])
        if with_deg:
            deg_v, degidx_v = deg_scratch
            pltpu.sync_copy(zeros_hbm.at[pl.ds(0, _DGR)], deg_v)
            pltpu.sync_copy(degidx_hbm, degidx_v)
            ones16 = jnp.ones((16,), jnp.float32)

        plsc.subcore_barrier()

        pltpu.async_copy(feat_hbm.at[sidx_v.at[0]], buf_a, sem_a)
        pltpu.async_copy(feat_hbm.at[sidx_v.at[1]], buf_b, sem_b)

        def count_deg(i):
            for j in range(_CH // 16):
                dv = dsts_v[i, pl.ds(16 * j, 16)]
                row = lax.shift_right_logical(dv, 7)
                col = lax.bitwise_and(dv, 127)
                plsc.addupdate_scatter(deg_v, [row, col], ones16)

        def half(i, buf, sem_g, slot, sem_i):
            pltpu.make_async_copy(feat_hbm.at[sidx_v.at[slot]], buf, sem_g).wait()

            @pl.when(i + 2 < _STEPS)
            def _():
                pltpu.async_copy(src_hbm.at[wid, i + 2], sidx_v.at[slot], sem_i)

            pltpu.sync_copy(buf, acc_sh.at[dsts_v.at[i]], add=True)
            if with_deg:
                count_deg(i)

            @pl.when(i + 2 < _STEPS)
            def _():
                pltpu.make_async_copy(src_hbm.at[wid, i + 2], sidx_v.at[slot],
                                      sem_i).wait()
                pltpu.async_copy(feat_hbm.at[sidx_v.at[slot]], buf, sem_g)

        def pair(k, carry):
            i0 = 2 * k
            half(i0, buf_a, sem_a, 0, sem_ia)
            half(i0 + 1, buf_b, sem_b, 1, sem_ib)
            return carry

        lax.fori_loop(0, _PAIRS, pair, 0)
        last = _STEPS - 1
        pltpu.make_async_copy(feat_hbm.at[sidx_v.at[0]], buf_a, sem_a).wait()
        pltpu.sync_copy(buf_a, acc_sh.at[dsts_v.at[last]], add=True)
        if with_deg:
            count_deg(last)
            pltpu.sync_copy(deg_v, acc_sh.at[degidx_v], add=True)

        plsc.subcore_barrier()

        pltpu.sync_copy(acc_sh.at[pl.ds(base_r, _ROWS_A)],
                        out_hbm.at[cid, pl.ds(base_r, _ROWS_A)])

        @pl.when(sid == _NS - 1)
        def _():
            t0 = pl.multiple_of(_NS * _ROWS_A, 8)
            pltpu.sync_copy(acc_sh.at[pl.ds(t0, tail)],
                            out_hbm.at[cid, pl.ds(t0, tail)])

    return seg_sum


_seg_sum_1 = _make_seg_sum(D_IN, True)
_seg_sum_2 = _make_seg_sum(H2, False)


_RB = 2000


def _layer1_body(msgp_ref, h_ref, deg_ref, w1_ref, b1_ref, w2_ref, y2_ref):
    msg = msgp_ref[0] + msgp_ref[1]
    agg = (msg + h_ref[...]) / (deg_ref[...] + 1.0)
    acc = jnp.dot(agg, w1_ref[...], preferred_element_type=jnp.float32)
    h1 = jnp.maximum(acc + b1_ref[...], 0.0)
    y2_ref[...] = jnp.dot(h1, w2_ref[...], preferred_element_type=jnp.float32)


def _layer1(msgp, x, deg, w1, b1, w2):
    return pl.pallas_call(
        _layer1_body,
        grid=(N // _RB,),
        in_specs=[
            pl.BlockSpec((_NC, _RB, D_IN), lambda i: (0, i, 0)),
            pl.BlockSpec((_RB, D_IN), lambda i: (i, 0)),
            pl.BlockSpec((_RB, 1), lambda i: (i, 0)),
            pl.BlockSpec((D_IN, D_IN), lambda i: (0, 0)),
            pl.BlockSpec((1, D_IN), lambda i: (0, 0)),
            pl.BlockSpec((D_IN, H2), lambda i: (0, 0)),
        ],
        out_specs=pl.BlockSpec((_RB, H2), lambda i: (i, 0)),
        out_shape=jax.ShapeDtypeStruct((N, H2), jnp.float32),
    )(msgp, x, deg, w1, b1.reshape(1, D_IN), w2)


def _cls_body(msgp_ref, y2_ref, deg_ref, b2_ref, wc_ref, bc_ref, o_ref):
    msg = msgp_ref[0] + msgp_ref[1]
    agg = (msg + y2_ref[...]) / (deg_ref[...] + 1.0)
    z = jnp.maximum(agg + b2_ref[...], 0.0)
    logits = jnp.dot(z, wc_ref[...], preferred_element_type=jnp.float32)
    logits = logits + bc_ref[...]
    m = jnp.max(logits, axis=-1, keepdims=True)
    e = jnp.exp(logits - m)
    o_ref[...] = e / jnp.sum(e, axis=-1, keepdims=True)


def _cls_layer(msgp, y2, deg, b2, wc, bc):
    return pl.pallas_call(
        _cls_body,
        grid=(N // _RB,),
        in_specs=[
            pl.BlockSpec((_NC, _RB, H2), lambda i: (0, i, 0)),
            pl.BlockSpec((_RB, H2), lambda i: (i, 0)),
            pl.BlockSpec((_RB, 1), lambda i: (i, 0)),
            pl.BlockSpec((1, H2), lambda i: (0, 0)),
            pl.BlockSpec((H2, NCLS), lambda i: (0, 0)),
            pl.BlockSpec((1, NCLS), lambda i: (0, 0)),
        ],
        out_specs=pl.BlockSpec((_RB, NCLS), lambda i: (i, 0)),
        out_shape=jax.ShapeDtypeStruct((N, NCLS), jnp.float32),
    )(msgp, y2, deg, b2.reshape(1, H2), wc, bc.reshape(1, NCLS))


_BM = 200


def _dec_body(a_ref, b_ref, o_ref):
    o_ref[...] = lax.dot_general(
        a_ref[...], b_ref[...],
        (((1,), (1,)), ((), ())),
        preferred_element_type=jnp.float32)


def _decoder(preds):
    return pl.pallas_call(
        _dec_body,
        grid=(N // _BM,),
        in_specs=[
            pl.BlockSpec((_BM, NCLS), lambda i: (i, 0)),
            pl.BlockSpec((N, NCLS), lambda i: (0, 0)),
        ],
        out_specs=pl.BlockSpec((_BM, N), lambda i: (i, 0)),
        out_shape=jax.ShapeDtypeStruct((N, N), jnp.float32),
    )(preds, preds)


def kernel(x, edge_index, W1, b1, W2, b2, Wc, bc):
    src = edge_index[0].astype(jnp.int32).reshape(_NW, _STEPS, _CH)
    dst = edge_index[1].astype(jnp.int32).reshape(_NW, _STEPS, _CH)
    zeros1 = jnp.zeros((_NROW, D_IN), jnp.float32)
    zeros2 = jnp.zeros((N, H2), jnp.float32)
    degidx = jnp.arange(N, N + _DGR, dtype=jnp.int32)

    msgp1 = _seg_sum_1(x, src, dst, zeros1, degidx)
    deg = (msgp1[0, N:] + msgp1[1, N:]).reshape(-1)[:N].reshape(N, 1)
    y2 = _layer1(msgp1, x, deg, W1, b1, W2)

    msgp2 = _seg_sum_2(y2, src, dst, zeros2, degidx)
    preds = _cls_layer(msgp2, y2, deg, b2, Wc, bc)

    adj_hat = _decoder(preds)
    return preds, adj_hat
